# baseline (speedup 1.0000x reference)
import jax
import jax.numpy as jnp
from jax import lax
from jax.experimental import pallas as pl
from jax.experimental.pallas import tpu as pltpu

N_NODES = 100000
D = 129
B = 4096


def _body(sidx_s, perm_s, mem_ref, vals_ref, out_ref, sem):
    del mem_ref

    def issue(k, n_issued):
        row = sidx_s[k]
        nxt = sidx_s[jnp.minimum(k + 1, B - 1)]
        is_winner = jnp.logical_or(k == B - 1, row != nxt)

        def do_issue():
            src = perm_s[k]
            pltpu.make_async_copy(
                vals_ref.at[pl.ds(src, 1)],
                out_ref.at[pl.ds(row, 1)],
                sem,
            ).start()

        pl.when(is_winner)(do_issue)
        return n_issued + is_winner.astype(jnp.int32)

    n_issued = lax.fori_loop(0, B, issue, jnp.int32(0))

    def drain(_, carry):
        pltpu.make_async_copy(
            vals_ref.at[pl.ds(0, 1)],
            out_ref.at[pl.ds(0, 1)],
            sem,
        ).wait()
        return carry

    lax.fori_loop(0, n_issued, drain, 0)


_call = pl.pallas_call(
    _body,
    grid_spec=pltpu.PrefetchScalarGridSpec(
        num_scalar_prefetch=2,
        grid=(1,),
        in_specs=[
            pl.BlockSpec(memory_space=pltpu.MemorySpace.HBM),
            pl.BlockSpec(memory_space=pltpu.MemorySpace.HBM),
        ],
        out_specs=pl.BlockSpec(memory_space=pltpu.MemorySpace.HBM),
        scratch_shapes=[pltpu.SemaphoreType.DMA],
    ),
    out_shape=jax.ShapeDtypeStruct((N_NODES, D), jnp.float32),
    input_output_aliases={2: 0},
)


def kernel(memory, node_idxs, values):
    idx = node_idxs.astype(jnp.int32)
    perm = jnp.argsort(idx, stable=True).astype(jnp.int32)
    sidx = idx[perm]
    return _call(sidx, perm, memory, values)
